# baseline (speedup 1.0000x reference)
import jax
import jax.numpy as jnp
import numpy as np
from jax.experimental import pallas as pl

N = 10000
NHID = 512
NOISE_ALPHA = 5.0


def _bn(h, g, be):
    mu = jnp.mean(h, axis=0)
    var = jnp.var(h, axis=0)
    return (h - mu) / jnp.sqrt(var + 1e-5) * g + be


def kernel(x, edge_index, W1, b1, W2, b2, W3, b3, g1, be1, g2, be2, g3, be3,
           Wp1, bp1, Wp2, bp2, Wd1, bd1, Wd2, bd2, Wc):
    loop = jnp.arange(N, dtype=edge_index.dtype)
    src = jnp.concatenate([edge_index[0], loop])
    dst = jnp.concatenate([edge_index[1], loop])
    deg = jax.ops.segment_sum(jnp.ones_like(dst, dtype=jnp.float32), dst, num_segments=N)
    dinv = jax.lax.rsqrt(jnp.maximum(deg, 1.0))
    norm = dinv[src] * dinv[dst]

    def gcn_layer(h, W, b):
        h = h @ W + b
        msg = h[src] * norm[:, None]
        return jax.ops.segment_sum(msg, dst, num_segments=N)

    h = jax.nn.relu(_bn(gcn_layer(x, W1, b1), g1, be1))
    h = jax.nn.relu(_bn(gcn_layer(h, W2, b2), g2, be2))
    z1 = jax.nn.relu(_bn(gcn_layer(h, W3, b3), g3, be3))

    mag = NOISE_ALPHA / jnp.sqrt(jnp.float32(NHID))
    noise = jax.random.uniform(jax.random.key(42), z1.shape, minval=-mag, maxval=mag, dtype=jnp.float32)
    z2 = z1 + noise

    def proj(z):
        return jax.nn.relu(z @ Wp1 + bp1) @ Wp2 + bp2

    z1p = proj(z1)
    z2p = proj(z2)
    pred1 = z1p @ Wc
    pred2 = z2p @ Wc
    d = z1p - z2p
    z = jax.nn.relu(d @ Wd1 + bd1) @ Wd2 + bd2
    return (pred1, pred2, z)


# speedup vs baseline: 2.0077x; 2.0077x over previous
import functools

import jax
import jax.numpy as jnp
from jax import lax
from jax.experimental import pallas as pl
from jax.experimental.pallas import tpu as pltpu
from jax.experimental.pallas import tpu_sc as plsc

_N = 10000
_E = 160000
_NHID = 512
_NOISE_ALPHA = 5.0
_K = 80
_NCHD = _E // 16 // _K
_RPS = 624
_AROWS = 640
_CC = 4
_EP = _E + _K


def _sc_mesh():
    return plsc.VectorSubcoreMesh(core_axis_name="c", subcore_axis_name="s")


def _row_part_copy(s, src_of, dst_of):
    base = s * _RPS
    pltpu.sync_copy(src_of(base, _RPS), dst_of(base, _RPS))

    @pl.when(s == 15)
    def _():
        pltpu.sync_copy(src_of(16 * _RPS, 16), dst_of(16 * _RPS, 16))


def _deg_body(dst_r, ones_hbm, out, dbuf, obuf, dacc):
    c = lax.axis_index("c")
    s = lax.axis_index("s")

    @pl.when(c == 0)
    def _():
        pltpu.sync_copy(dst_r.at[s], dbuf)
        pltpu.sync_copy(ones_hbm.at[pl.ds(0, _K)], obuf)
        _row_part_copy(s, lambda o, n: ones_hbm.at[pl.ds(0, n)],
                       lambda o, n: dacc.at[pl.ds(o, n)])
        plsc.subcore_barrier()

        @pl.loop(0, _NCHD)
        def _(j):
            pltpu.sync_copy(obuf, dacc.at[dbuf.at[j]], add=True)

        plsc.subcore_barrier()
        _row_part_copy(s, lambda o, n: dacc.at[pl.ds(o, n)],
                       lambda o, n: out.at[pl.ds(o, n)])


@functools.partial(
    pl.kernel,
    out_type=jax.ShapeDtypeStruct((_N, 128), jnp.float32),
    mesh=_sc_mesh(),
    scratch_types=[
        pltpu.VMEM((_NCHD, _K), jnp.int32),
        pltpu.VMEM((_K, 128), jnp.float32),
        pltpu.VMEM_SHARED((_N, 128), jnp.float32),
    ],
)
def _sc_deg(dst_r, ones_hbm, out, dbuf, obuf, dacc):
    _deg_body(dst_r, ones_hbm, out, dbuf, obuf, dacc)


def _agg_body(hs_flat, psrc, pdst, starts_h, dinv_h, out,
              sbuf, dbuf, rbuf, acc, dinv_v, starts_v, gsem):
    c = lax.axis_index("c")
    s = lax.axis_index("s")
    base = s * _RPS

    pltpu.sync_copy(dinv_h, dinv_v)
    pltpu.sync_copy(starts_h, starts_v)
    sv16 = starts_v[pl.ds(s, 16)]
    start = sv16[0]
    end = sv16[1]
    s8 = (start // 8) * 8
    nch = (end - s8 + (_K - 1)) // _K

    def process(j, b, cc):
        @pl.loop(0, _K)
        def _(e):
            dv = dbuf[b, pl.ds(e, 16)][0]
            sv = jnp.clip(sbuf[b, pl.ds(e, 16)][0] - cc * _N, 0, _N - 1)
            nrm = dinv_v[pl.ds(sv, 16)][0] * dinv_v[pl.ds(dv, 16)][0]
            eg = s8 + j * _K + e
            valid = jnp.logical_and(eg >= start, eg < end)
            nrm = jnp.where(valid, nrm, 0.0)
            dl = jnp.clip(dv - base, 0, _AROWS - 1)
            nv = jnp.broadcast_to(nrm, (16,))
            for k in range(8):
                acc[dl, pl.ds(16 * k, 16)] += nv * rbuf[b, e, pl.ds(16 * k, 16)]

    def idx_dma(j, b, cc):
        off = s8 + j * _K
        pltpu.sync_copy(psrc.at[pl.ds(cc * _EP + off, _K)],
                        sbuf.at[b, pl.ds(0, _K)])
        pltpu.sync_copy(pdst.at[pl.ds(off, _K)], dbuf.at[b, pl.ds(0, _K)])

    def gather(b):
        idx = sbuf.at[b, pl.ds(0, _K)]
        pltpu.async_copy(hs_flat.at[idx], rbuf.at[b], gsem.at[b])

    def gwait(b):
        idx = sbuf.at[b, pl.ds(0, _K)]
        pltpu.make_async_copy(hs_flat.at[idx], rbuf.at[b], gsem.at[b]).wait()

    for p in range(2):
        cc = c * 2 + p

        zv = jnp.zeros((16,), jnp.float32)

        @pl.loop(0, _AROWS)
        def _(i):
            for k in range(8):
                acc[i, pl.ds(16 * k, 16)] = zv

        @pl.when(nch > 0)
        def _():
            idx_dma(0, 0, cc)
            gather(0)

        @pl.when(nch > 1)
        def _():
            idx_dma(1, 1, cc)
            gather(1)

        npair = (nch + 1) // 2

        @pl.loop(0, npair)
        def _(t):
            j0 = 2 * t

            @pl.when(j0 < nch)
            def _():
                gwait(0)
                process(j0, 0, cc)

                @pl.when(j0 + 2 < nch)
                def _():
                    idx_dma(j0 + 2, 0, cc)
                    gather(0)

            @pl.when(j0 + 1 < nch)
            def _():
                gwait(1)
                process(j0 + 1, 1, cc)

                @pl.when(j0 + 3 < nch)
                def _():
                    idx_dma(j0 + 3, 1, cc)
                    gather(1)

        def self_rows(off, n):
            pltpu.sync_copy(hs_flat.at[pl.ds(cc * _N + base + off, n)],
                            rbuf.at[0, pl.ds(0, n)])

            @pl.loop(0, n)
            def _(r):
                dd = dinv_v[pl.ds(base + off + r, 16)][0]
                nrm = dd * dd
                nv = jnp.broadcast_to(nrm, (16,))
                for k in range(8):
                    acc[off + r, pl.ds(16 * k, 16)] += (
                        nv * rbuf[0, r, pl.ds(16 * k, 16)])

        for i in range(7):
            self_rows(i * _K, _K)
        self_rows(7 * _K, 64)

        @pl.when(s == 15)
        def _():
            self_rows(_RPS, 16)

        pltpu.sync_copy(acc.at[pl.ds(0, _RPS)],
                        out.at[pl.ds(base, _RPS), pl.ds(cc * 128, 128)])

        @pl.when(s == 15)
        def _():
            pltpu.sync_copy(acc.at[pl.ds(_RPS, 16)],
                            out.at[pl.ds(16 * _RPS, 16), pl.ds(cc * 128, 128)])


@functools.partial(
    pl.kernel,
    out_type=jax.ShapeDtypeStruct((_N, _NHID), jnp.float32),
    mesh=_sc_mesh(),
    scratch_types=[
        pltpu.VMEM((2, _K + 16), jnp.int32),
        pltpu.VMEM((2, _K + 16), jnp.int32),
        pltpu.VMEM((2, _K, 128), jnp.float32),
        pltpu.VMEM((_AROWS, 128), jnp.float32),
        pltpu.VMEM((_N + 16,), jnp.float32),
        pltpu.VMEM((32,), jnp.int32),
        pltpu.SemaphoreType.DMA((2,)),
    ],
)
def _sc_agg(hs_flat, psrc, pdst, starts_h, dinv_h, out,
            sbuf, dbuf, rbuf, acc, dinv_v, starts_v, gsem):
    _agg_body(hs_flat, psrc, pdst, starts_h, dinv_h, out,
              sbuf, dbuf, rbuf, acc, dinv_v, starts_v, gsem)


_BM = 1000


def _mm_chunked(x, w, b):
    m, kd = x.shape

    def body(x_ref, w_ref, b_ref, o_ref):
        y = jnp.dot(x_ref[...], w_ref[...], preferred_element_type=jnp.float32)
        o_ref[0] = y + b_ref[...]

    return pl.pallas_call(
        body,
        grid=(m // _BM, _CC),
        in_specs=[
            pl.BlockSpec((_BM, kd), lambda i, j: (i, 0)),
            pl.BlockSpec((kd, 128), lambda i, j: (0, j)),
            pl.BlockSpec((1, 128), lambda i, j: (0, j)),
        ],
        out_specs=pl.BlockSpec((1, _BM, 128), lambda i, j: (j, i, 0)),
        out_shape=jax.ShapeDtypeStruct((_CC, m, 128), jnp.float32),
    )(x, w, b.reshape(1, -1))


def _mm(x, w, b=None, relu=False, diff=False):
    m, kd = x.shape
    nd = w.shape[1]
    mo = m // 2 if diff else m

    def body(*refs):
        if diff:
            xv = refs[0][...] - refs[1][...]
            rest = refs[2:]
        else:
            xv = refs[0][...]
            rest = refs[1:]
        y = jnp.dot(xv, rest[0][...], preferred_element_type=jnp.float32)
        if b is not None:
            y = y + rest[1][...]
        o_ref = rest[-1]
        o_ref[...] = jnp.maximum(y, 0.0) if relu else y

    nblk = mo // _BM
    in_specs = [pl.BlockSpec((_BM, kd), lambda i: (i, 0))]
    args = [x]
    if diff:
        in_specs.append(pl.BlockSpec((_BM, kd), lambda i: (i + nblk, 0)))
        args.append(x)
    in_specs.append(pl.BlockSpec((kd, nd), lambda i: (0, 0)))
    args.append(w)
    if b is not None:
        in_specs.append(pl.BlockSpec((1, nd), lambda i: (0, 0)))
        args.append(b.reshape(1, -1))

    return pl.pallas_call(
        body,
        grid=(nblk,),
        in_specs=in_specs,
        out_specs=pl.BlockSpec((_BM, nd), lambda i: (i, 0)),
        out_shape=jax.ShapeDtypeStruct((mo, nd), jnp.float32),
    )(*args)


def _bn_apply(agg, mu, var, g, be):
    def body(a_ref, m_ref, v_ref, g_ref, be_ref, o_ref):
        h = ((a_ref[...] - m_ref[...]) / jnp.sqrt(v_ref[...] + 1e-5)
             * g_ref[...] + be_ref[...])
        o_ref[...] = jnp.maximum(h, 0.0)

    return pl.pallas_call(
        body,
        grid=(_N // _BM,),
        in_specs=[
            pl.BlockSpec((_BM, _NHID), lambda i: (i, 0)),
            pl.BlockSpec((1, _NHID), lambda i: (0, 0)),
            pl.BlockSpec((1, _NHID), lambda i: (0, 0)),
            pl.BlockSpec((1, _NHID), lambda i: (0, 0)),
            pl.BlockSpec((1, _NHID), lambda i: (0, 0)),
        ],
        out_specs=pl.BlockSpec((_BM, _NHID), lambda i: (i, 0)),
        out_shape=jax.ShapeDtypeStruct((_N, _NHID), jnp.float32),
    )(agg, mu.reshape(1, -1), var.reshape(1, -1),
      g.reshape(1, -1), be.reshape(1, -1))


def _bn_apply_noise(agg, mu, var, g, be, noise):
    def body(a_ref, m_ref, v_ref, g_ref, be_ref, n_ref, o_ref):
        i = pl.program_id(0)
        h = ((a_ref[...] - m_ref[...]) / jnp.sqrt(v_ref[...] + 1e-5)
             * g_ref[...] + be_ref[...])
        h = jnp.maximum(h, 0.0)
        o_ref[...] = h + (i >= _N // _BM).astype(jnp.float32) * n_ref[...]

    nblk = _N // _BM
    return pl.pallas_call(
        body,
        grid=(2 * nblk,),
        in_specs=[
            pl.BlockSpec((_BM, _NHID), lambda i: (lax.rem(i, nblk), 0)),
            pl.BlockSpec((1, _NHID), lambda i: (0, 0)),
            pl.BlockSpec((1, _NHID), lambda i: (0, 0)),
            pl.BlockSpec((1, _NHID), lambda i: (0, 0)),
            pl.BlockSpec((1, _NHID), lambda i: (0, 0)),
            pl.BlockSpec((_BM, _NHID), lambda i: (lax.rem(i, nblk), 0)),
        ],
        out_specs=pl.BlockSpec((_BM, _NHID), lambda i: (i, 0)),
        out_shape=jax.ShapeDtypeStruct((2 * _N, _NHID), jnp.float32),
    )(agg, mu.reshape(1, -1), var.reshape(1, -1),
      g.reshape(1, -1), be.reshape(1, -1), noise)


def kernel(x, edge_index, W1, b1, W2, b2, W3, b3, g1, be1, g2, be2, g3, be3,
           Wp1, bp1, Wp2, bp2, Wd1, bd1, Wd2, bd2, Wc):
    src = edge_index[0]
    dst = edge_index[1]

    order = jnp.argsort(dst, stable=True)
    s_src = src[order]
    s_dst = dst[order]
    offs = (jnp.arange(_CC, dtype=jnp.int32) * _N)[:, None]
    psrc = jnp.pad(s_src[None, :] + offs, ((0, 0), (0, _K))).reshape(-1)
    pdst = jnp.pad(s_dst, (0, _K))
    bounds = jnp.arange(17, dtype=jnp.int32) * _RPS
    starts = jnp.searchsorted(s_dst, bounds.astype(s_dst.dtype), side="left")
    starts = starts.at[16].set(_E)
    starts = jnp.pad(starts.astype(jnp.int32), (0, 15))

    dst_r = dst.reshape(16, _NCHD, _K)
    ones_c = jnp.ones((_RPS, 128), jnp.float32)
    deg128 = _sc_deg(dst_r, ones_c)
    dinv = jnp.pad(lax.rsqrt(jnp.maximum(deg128[:, 0], 1.0)), (0, 16))

    def gcn_layer(h, W, b):
        hs4 = _mm_chunked(h, W, b).reshape(_CC * _N, 128)
        agg = _sc_agg(hs4, psrc, pdst, starts, dinv)
        mu = jnp.mean(agg, axis=0)
        var = jnp.var(agg, axis=0)
        return agg, mu, var

    agg, mu, var = gcn_layer(x, W1, b1)
    h = _bn_apply(agg, mu, var, g1, be1)
    agg, mu, var = gcn_layer(h, W2, b2)
    h = _bn_apply(agg, mu, var, g2, be2)
    agg, mu, var = gcn_layer(h, W3, b3)

    mag = _NOISE_ALPHA / jnp.sqrt(jnp.float32(_NHID))
    noise = jax.random.uniform(jax.random.key(42), (_N, _NHID),
                               minval=-mag, maxval=mag, dtype=jnp.float32)
    Z = _bn_apply_noise(agg, mu, var, g3, be3, noise)

    P = _mm(Z, Wp1, bp1, relu=True)
    ZPP = _mm(P, Wp2, bp2)
    Wc_pad = jnp.pad(Wc, ((0, 0), (0, 128 - Wc.shape[1])))
    preds = _mm(ZPP, Wc_pad)
    T = _mm(ZPP, Wd1, bd1, relu=True, diff=True)
    zout = _mm(T, Wd2, bd2)

    pred1 = preds[:_N, :40]
    pred2 = preds[_N:, :40]
    return (pred1, pred2, zout)
